# baseline (speedup 1.0000x reference)
import functools

import jax
import jax.numpy as jnp
from jax import lax
from jax.experimental import pallas as pl
from jax.experimental.pallas import tpu as pltpu
from jax.experimental.pallas import tpu_sc as plsc

_VOCAB = 100000
_EMBED = 64
_HIDDEN = 128
_BATCH = 4096
_SEQ = 50

_NC = 2
_NS = 16
_NW = _NC * _NS
_ROWS_PER_W = _BATCH // _NW
_CHUNK_ROWS = 2
_CHUNK_IDX = _CHUNK_ROWS * _SEQ
_NCHUNK = _ROWS_PER_W // _CHUNK_ROWS


def _pool_body(src_r, table, out, idx_v, rows_v, acc_v, sem):
    wid = lax.axis_index("s") * _NC + lax.axis_index("c")
    pltpu.sync_copy(src_r.at[wid], idx_v)

    def chunk(c, carry):
        pltpu.async_copy(table.at[idx_v.at[c]], rows_v, sem).wait()
        for r in range(_CHUNK_ROWS):
            for k in range(_EMBED // 16):
                acc = rows_v[r * _SEQ, pl.ds(16 * k, 16)]
                for s in range(1, _SEQ):
                    acc = acc + rows_v[r * _SEQ + s, pl.ds(16 * k, 16)]
                acc_v[_CHUNK_ROWS * c + r, pl.ds(16 * k, 16)] = acc * (1.0 / _SEQ)
        return carry

    lax.fori_loop(0, _NCHUNK, chunk, 0)
    pltpu.sync_copy(acc_v, out.at[pl.ds(wid * _ROWS_PER_W, _ROWS_PER_W)])


_pool_call = functools.partial(
    pl.kernel,
    mesh=plsc.VectorSubcoreMesh(core_axis_name="c", subcore_axis_name="s"),
    out_type=jax.ShapeDtypeStruct((_BATCH, _EMBED), jnp.float32),
    scratch_types=[
        pltpu.VMEM((_NCHUNK, _CHUNK_IDX), jnp.int32),
        pltpu.VMEM((_CHUNK_IDX, _EMBED), jnp.float32),
        pltpu.VMEM((_ROWS_PER_W, _EMBED), jnp.float32),
        pltpu.SemaphoreType.DMA,
    ],
    compiler_params=pltpu.CompilerParams(use_tc_tiling_on_sc=False),
)(_pool_body)


_BN = 512
_NSTEPS = _VOCAB // _BN
_TAIL = _VOCAB - _NSTEPS * _BN
_RING = 4


def _mlp_body(
    pooled_ref, W1_ref, b1_ref, W2_ref, b2_ref, out_hbm,
    h_ref, ob0, ob1, ob2, ob3, sems,
):
    j = pl.program_id(0)
    obufs = [ob0, ob1, ob2, ob3]

    @pl.when(j == 0)
    def _():
        h = jnp.dot(pooled_ref[...], W1_ref[...], preferred_element_type=jnp.float32)
        h_ref[...] = jnp.maximum(h + b1_ref[...], 0.0).astype(jnp.bfloat16)

    for s in range(_RING):
        @pl.when((j % _RING == s) & (j >= _RING))
        def _(s=s):
            pltpu.make_async_copy(
                obufs[s],
                out_hbm.at[:, pl.ds((j - _RING) * _BN, _BN)],
                sems.at[s],
            ).wait()

    w2 = W2_ref[...].astype(jnp.bfloat16)

    for s in range(_RING):
        @pl.when(j % _RING == s)
        def _(s=s):
            obufs[s][...] = (
                jnp.dot(h_ref[...], w2, preferred_element_type=jnp.float32)
                + b2_ref[...]
            )
            pltpu.make_async_copy(
                obufs[s],
                out_hbm.at[:, pl.ds(j * _BN, _BN)],
                sems.at[s],
            ).start(priority=s % 2)

    @pl.when(j == _NSTEPS - 1)
    def _():
        for jj in range(_NSTEPS - _RING, _NSTEPS):
            pltpu.make_async_copy(
                obufs[jj % _RING],
                out_hbm.at[:, pl.ds(jj * _BN, _BN)],
                sems.at[jj % _RING],
            ).wait()


_mlp_call = pl.pallas_call(
    _mlp_body,
    grid=(_NSTEPS,),
    in_specs=[
        pl.BlockSpec((_BATCH, _EMBED), lambda j: (0, 0)),
        pl.BlockSpec((_EMBED, _HIDDEN), lambda j: (0, 0)),
        pl.BlockSpec((1, _HIDDEN), lambda j: (0, 0)),
        pl.BlockSpec((_HIDDEN, _BN), lambda j: (0, j)),
        pl.BlockSpec((1, _BN), lambda j: (0, j)),
    ],
    out_specs=pl.BlockSpec(memory_space=pl.ANY),
    out_shape=jax.ShapeDtypeStruct((_BATCH, _VOCAB), jnp.float32),
    scratch_shapes=[
        pltpu.VMEM((_BATCH, _HIDDEN), jnp.bfloat16),
        pltpu.VMEM((_BATCH, _BN), jnp.float32),
        pltpu.VMEM((_BATCH, _BN), jnp.float32),
        pltpu.VMEM((_BATCH, _BN), jnp.float32),
        pltpu.VMEM((_BATCH, _BN), jnp.float32),
        pltpu.SemaphoreType.DMA((_RING,)),
    ],
)


def _tail_body(out_in, pooled_ref, W1_ref, b1_ref, W2_ref, b2_ref, out_ref):
    del out_in
    h = jnp.dot(pooled_ref[...], W1_ref[...], preferred_element_type=jnp.float32)
    h = jnp.maximum(h + b1_ref[...], 0.0).astype(jnp.bfloat16)
    w2 = W2_ref[...].astype(jnp.bfloat16)
    out_ref[...] = (
        jnp.dot(h, w2, preferred_element_type=jnp.float32) + b2_ref[...]
    )


_TAIL_W = 640
_TAIL_BLK = _NSTEPS * _BN // _TAIL_W

_tail_call = pl.pallas_call(
    _tail_body,
    grid=(1,),
    in_specs=[
        pl.BlockSpec(memory_space=pl.ANY),
        pl.BlockSpec((_BATCH, _EMBED), lambda i: (0, 0)),
        pl.BlockSpec((_EMBED, _HIDDEN), lambda i: (0, 0)),
        pl.BlockSpec((1, _HIDDEN), lambda i: (0, 0)),
        pl.BlockSpec((_HIDDEN, _TAIL_W), lambda i: (0, _TAIL_BLK)),
        pl.BlockSpec((1, _TAIL_W), lambda i: (0, _TAIL_BLK)),
    ],
    out_specs=pl.BlockSpec((_BATCH, _TAIL_W), lambda i: (0, _TAIL_BLK)),
    out_shape=jax.ShapeDtypeStruct((_BATCH, _VOCAB), jnp.float32),
    input_output_aliases={0: 0},
)


def kernel(src, emb_table, W1, b1, W2, b2):
    src_r = src.reshape(_NW, _NCHUNK, _CHUNK_IDX).astype(jnp.int32)
    pooled = _pool_call(src_r, emb_table)
    b1r, b2r = b1.reshape(1, -1), b2.reshape(1, -1)
    out = _mlp_call(pooled, W1, b1r, W2, b2r)
    return _tail_call(out, pooled, W1, b1r, W2, b2r)


# speedup vs baseline: 2.9941x; 2.9941x over previous
import functools

import jax
import jax.numpy as jnp
from jax import lax
from jax.experimental import pallas as pl
from jax.experimental.pallas import tpu as pltpu
from jax.experimental.pallas import tpu_sc as plsc

_VOCAB = 100000
_EMBED = 64
_HIDDEN = 128
_BATCH = 4096
_SEQ = 50

_NC = 2
_NS = 16
_NW = _NC * _NS
_ROWS_PER_W = _BATCH // _NW
_CHUNK_ROWS = 2
_CHUNK_IDX = _CHUNK_ROWS * _SEQ
_NCHUNK = _ROWS_PER_W // _CHUNK_ROWS


def _pool_body(src_r, table, out, idx_v, rows_v, acc_v, sem):
    wid = lax.axis_index("s") * _NC + lax.axis_index("c")
    pltpu.sync_copy(src_r.at[wid], idx_v)

    def chunk(c, carry):
        pltpu.async_copy(table.at[idx_v.at[c]], rows_v, sem).wait()
        for r in range(_CHUNK_ROWS):
            for k in range(_EMBED // 16):
                acc = rows_v[r * _SEQ, pl.ds(16 * k, 16)]
                for s in range(1, _SEQ):
                    acc = acc + rows_v[r * _SEQ + s, pl.ds(16 * k, 16)]
                acc_v[_CHUNK_ROWS * c + r, pl.ds(16 * k, 16)] = acc * (1.0 / _SEQ)
        return carry

    lax.fori_loop(0, _NCHUNK, chunk, 0)
    pltpu.sync_copy(acc_v, out.at[pl.ds(wid * _ROWS_PER_W, _ROWS_PER_W)])


_pool_call = functools.partial(
    pl.kernel,
    mesh=plsc.VectorSubcoreMesh(core_axis_name="c", subcore_axis_name="s"),
    out_type=jax.ShapeDtypeStruct((_BATCH, _EMBED), jnp.float32),
    scratch_types=[
        pltpu.VMEM((_NCHUNK, _CHUNK_IDX), jnp.int32),
        pltpu.VMEM((_CHUNK_IDX, _EMBED), jnp.float32),
        pltpu.VMEM((_ROWS_PER_W, _EMBED), jnp.float32),
        pltpu.SemaphoreType.DMA,
    ],
    compiler_params=pltpu.CompilerParams(use_tc_tiling_on_sc=False),
)(_pool_body)


_BN = 512
_NT = pl.cdiv(_VOCAB, _BN)
_TAIL = _VOCAB - (_NT - 1) * _BN
_RING = 4


def _mlp_body(
    pooled_ref, W1_ref, b1_ref, W2T_ref, b2_ref, outT_hbm,
    h_ref, ob0, ob1, ob2, ob3, sems,
):
    j = pl.program_id(0)
    obufs = [ob0, ob1, ob2, ob3]

    @pl.when(j == 0)
    def _():
        h = jnp.dot(pooled_ref[...], W1_ref[...], preferred_element_type=jnp.float32)
        h_ref[...] = jnp.maximum(h + b1_ref[...], 0.0).astype(jnp.bfloat16)

    for s in range(_RING):
        @pl.when((j % _RING == s) & (j >= _RING))
        def _(s=s):
            pltpu.make_async_copy(
                obufs[s],
                outT_hbm.at[pl.ds((j - _RING) * _BN, _BN)],
                sems.at[s],
            ).wait()

    w2t = W2T_ref[...].astype(jnp.bfloat16)

    def _block_val():
        acc = jax.lax.dot_general(
            w2t, h_ref[...],
            dimension_numbers=(((1,), (1,)), ((), ())),
            preferred_element_type=jnp.float32,
        )
        return acc + b2_ref[...]

    for s in range(_RING):
        @pl.when((j % _RING == s) & (j < _NT - 1))
        def _(s=s):
            obufs[s][...] = _block_val()
            pltpu.make_async_copy(
                obufs[s],
                outT_hbm.at[pl.ds(j * _BN, _BN)],
                sems.at[s],
            ).start(priority=s % 2)

    @pl.when(j == _NT - 1)
    def _():
        sl = (_NT - 1) % _RING
        obufs[sl][...] = _block_val()
        tail_copy = pltpu.make_async_copy(
            obufs[sl].at[pl.ds(0, _TAIL)],
            outT_hbm.at[pl.ds((_NT - 1) * _BN, _TAIL)],
            sems.at[sl],
        )
        tail_copy.start(priority=sl % 2)
        for jj in range(_NT - _RING, _NT - 1):
            pltpu.make_async_copy(
                obufs[jj % _RING],
                outT_hbm.at[pl.ds(jj * _BN, _BN)],
                sems.at[jj % _RING],
            ).wait()
        tail_copy.wait()


_mlp_call = pl.pallas_call(
    _mlp_body,
    grid=(_NT,),
    in_specs=[
        pl.BlockSpec((_BATCH, _EMBED), lambda j: (0, 0)),
        pl.BlockSpec((_EMBED, _HIDDEN), lambda j: (0, 0)),
        pl.BlockSpec((1, _HIDDEN), lambda j: (0, 0)),
        pl.BlockSpec((_BN, _HIDDEN), lambda j: (j, 0)),
        pl.BlockSpec((_BN, 1), lambda j: (j, 0)),
    ],
    out_specs=pl.BlockSpec(memory_space=pl.ANY),
    out_shape=jax.ShapeDtypeStruct((_VOCAB, _BATCH), jnp.float32),
    scratch_shapes=[
        pltpu.VMEM((_BATCH, _HIDDEN), jnp.bfloat16),
        pltpu.VMEM((_BN, _BATCH), jnp.float32),
        pltpu.VMEM((_BN, _BATCH), jnp.float32),
        pltpu.VMEM((_BN, _BATCH), jnp.float32),
        pltpu.VMEM((_BN, _BATCH), jnp.float32),
        pltpu.SemaphoreType.DMA((_RING,)),
    ],
)


def kernel(src, emb_table, W1, b1, W2, b2):
    src_r = src.reshape(_NW, _NCHUNK, _CHUNK_IDX).astype(jnp.int32)
    pooled = _pool_call(src_r, emb_table)
    outT = _mlp_call(pooled, W1, b1.reshape(1, -1), W2.T, b2.reshape(-1, 1))
    return outT.T


# speedup vs baseline: 3.0401x; 1.0154x over previous
import functools

import jax
import jax.numpy as jnp
from jax import lax
from jax.experimental import pallas as pl
from jax.experimental.pallas import tpu as pltpu
from jax.experimental.pallas import tpu_sc as plsc

_VOCAB = 100000
_EMBED = 64
_HIDDEN = 128
_BATCH = 4096
_SEQ = 50

_NC = 2
_NS = 16
_NW = _NC * _NS
_ROWS_PER_W = _BATCH // _NW
_CHUNK_ROWS = 2
_CHUNK_IDX = _CHUNK_ROWS * _SEQ
_NCHUNK = _ROWS_PER_W // _CHUNK_ROWS


def _pool_body(src_r, table, out, idx_v, rows_v0, rows_v1, acc_v, sem0, sem1):
    wid = lax.axis_index("s") * _NC + lax.axis_index("c")
    pltpu.sync_copy(src_r.at[wid], idx_v)

    rows = [rows_v0, rows_v1]
    sems = [sem0, sem1]

    pltpu.async_copy(table.at[idx_v.at[0]], rows_v0, sem0)

    def _pool_chunk(c, buf):
        for r in range(_CHUNK_ROWS):
            for k in range(_EMBED // 16):
                acc = buf[r * _SEQ, pl.ds(16 * k, 16)]
                for s in range(1, _SEQ):
                    acc = acc + buf[r * _SEQ + s, pl.ds(16 * k, 16)]
                acc_v[_CHUNK_ROWS * c + r, pl.ds(16 * k, 16)] = acc * (1.0 / _SEQ)

    def chunk(c, carry):
        for b in range(2):
            @pl.when(c % 2 == b)
            def _(b=b):
                pltpu.make_async_copy(table.at[idx_v.at[c]], rows[b], sems[b]).wait()
                @pl.when(c < _NCHUNK - 1)
                def _():
                    pltpu.async_copy(
                        table.at[idx_v.at[c + 1]], rows[1 - b], sems[1 - b]
                    )
                _pool_chunk(c, rows[b])
        return carry

    lax.fori_loop(0, _NCHUNK, chunk, 0)
    pltpu.sync_copy(acc_v, out.at[pl.ds(wid * _ROWS_PER_W, _ROWS_PER_W)])


_pool_call = functools.partial(
    pl.kernel,
    mesh=plsc.VectorSubcoreMesh(core_axis_name="c", subcore_axis_name="s"),
    out_type=jax.ShapeDtypeStruct((_BATCH, _EMBED), jnp.float32),
    scratch_types=[
        pltpu.VMEM((_NCHUNK, _CHUNK_IDX), jnp.int32),
        pltpu.VMEM((_CHUNK_IDX, _EMBED), jnp.float32),
        pltpu.VMEM((_CHUNK_IDX, _EMBED), jnp.float32),
        pltpu.VMEM((_ROWS_PER_W, _EMBED), jnp.float32),
        pltpu.SemaphoreType.DMA,
        pltpu.SemaphoreType.DMA,
    ],
    compiler_params=pltpu.CompilerParams(use_tc_tiling_on_sc=False),
)(_pool_body)


_BN = 512
_NT = pl.cdiv(_VOCAB, _BN)
_TAIL = _VOCAB - (_NT - 1) * _BN
_RING = 4


def _mlp_body(
    pooled_ref, W1_ref, b1_ref, W2T_ref, b2_ref, outT_hbm,
    h_ref, ob0, ob1, ob2, ob3, sems,
):
    j = pl.program_id(0)
    obufs = [ob0, ob1, ob2, ob3]

    @pl.when(j == 0)
    def _():
        h = jnp.dot(pooled_ref[...], W1_ref[...], preferred_element_type=jnp.float32)
        h_ref[...] = jnp.maximum(h + b1_ref[...], 0.0).astype(jnp.bfloat16)

    for s in range(_RING):
        @pl.when((j % _RING == s) & (j >= _RING))
        def _(s=s):
            pltpu.make_async_copy(
                obufs[s],
                outT_hbm.at[pl.ds((j - _RING) * _BN, _BN)],
                sems.at[s],
            ).wait()

    w2t = W2T_ref[...].astype(jnp.bfloat16)

    def _block_val():
        acc = jax.lax.dot_general(
            w2t, h_ref[...],
            dimension_numbers=(((1,), (1,)), ((), ())),
            preferred_element_type=jnp.float32,
        )
        return acc + b2_ref[...]

    for s in range(_RING):
        @pl.when((j % _RING == s) & (j < _NT - 1))
        def _(s=s):
            obufs[s][...] = _block_val()
            pltpu.make_async_copy(
                obufs[s],
                outT_hbm.at[pl.ds(j * _BN, _BN)],
                sems.at[s],
            ).start(priority=s % 2)

    @pl.when(j == _NT - 1)
    def _():
        sl = (_NT - 1) % _RING
        obufs[sl][...] = _block_val()
        tail_copy = pltpu.make_async_copy(
            obufs[sl].at[pl.ds(0, _TAIL)],
            outT_hbm.at[pl.ds((_NT - 1) * _BN, _TAIL)],
            sems.at[sl],
        )
        tail_copy.start(priority=sl % 2)
        for jj in range(_NT - _RING, _NT - 1):
            pltpu.make_async_copy(
                obufs[jj % _RING],
                outT_hbm.at[pl.ds(jj * _BN, _BN)],
                sems.at[jj % _RING],
            ).wait()
        tail_copy.wait()


_mlp_call = pl.pallas_call(
    _mlp_body,
    grid=(_NT,),
    in_specs=[
        pl.BlockSpec((_BATCH, _EMBED), lambda j: (0, 0)),
        pl.BlockSpec((_EMBED, _HIDDEN), lambda j: (0, 0)),
        pl.BlockSpec((1, _HIDDEN), lambda j: (0, 0)),
        pl.BlockSpec((_BN, _HIDDEN), lambda j: (j, 0)),
        pl.BlockSpec((_BN, 1), lambda j: (j, 0)),
    ],
    out_specs=pl.BlockSpec(memory_space=pl.ANY),
    out_shape=jax.ShapeDtypeStruct((_VOCAB, _BATCH), jnp.float32),
    scratch_shapes=[
        pltpu.VMEM((_BATCH, _HIDDEN), jnp.bfloat16),
        pltpu.VMEM((_BN, _BATCH), jnp.float32),
        pltpu.VMEM((_BN, _BATCH), jnp.float32),
        pltpu.VMEM((_BN, _BATCH), jnp.float32),
        pltpu.VMEM((_BN, _BATCH), jnp.float32),
        pltpu.SemaphoreType.DMA((_RING,)),
    ],
)


def kernel(src, emb_table, W1, b1, W2, b2):
    src_r = src.reshape(_NW, _NCHUNK, _CHUNK_IDX).astype(jnp.int32)
    pooled = _pool_call(src_r, emb_table)
    outT = _mlp_call(pooled, W1, b1.reshape(1, -1), W2.T, b2.reshape(-1, 1))
    return outT.T
